# Initial kernel scaffold; baseline (speedup 1.0000x reference)
import functools

import jax
import jax.numpy as jnp
from jax import lax
from jax.experimental import pallas as pl
from jax.experimental.pallas import tpu as pltpu
from jax.experimental.pallas import tpu_sc as plsc

N = 10000
E = 320000
D = 128
NCORE = 2
NSUB = 16
LANES = 16
CHUNK = 80
EPT = E // NSUB
NCHUNK = EPT // CHUNK
ROWS_PT = N // NSUB


def _seg_body(x_hbm, src_hbm, dst_hbm, w_hbm, zero_hbm, out_hbm,
              srcv, dstv, wv, rows, acc, sem):
    cid = lax.axis_index("c")
    sid = lax.axis_index("s")
    row0 = sid * ROWS_PT
    pltpu.sync_copy(zero_hbm.at[pl.ds(row0, ROWS_PT)],
                    acc.at[pl.ds(row0, ROWS_PT)])
    crow0 = sid * NCHUNK
    pltpu.sync_copy(src_hbm.at[cid, pl.ds(crow0, NCHUNK)], srcv)
    pltpu.sync_copy(dst_hbm.at[cid, pl.ds(crow0, NCHUNK)], dstv)
    pltpu.sync_copy(w_hbm.at[cid, pl.ds(crow0, NCHUNK)], wv)
    plsc.subcore_barrier()

    @pl.loop(0, NCHUNK)
    def _(c):
        pltpu.async_copy(x_hbm.at[srcv.at[c]], rows, sem).wait()

        @pl.loop(0, CHUNK)
        def _(e):
            w = wv[c, e]
            for j in range(D // LANES):
                sl = (e, pl.ds(j * LANES, LANES))
                rows[sl] = rows[sl] * w

        pltpu.sync_copy(rows, acc.at[dstv.at[c]], add=True)

    plsc.subcore_barrier()
    pltpu.sync_copy(acc.at[pl.ds(row0, ROWS_PT)],
                    out_hbm.at[cid, pl.ds(row0, ROWS_PT)])


def _sc_aggregate(x, src2, dst2, w2, zeros):
    mesh = plsc.VectorSubcoreMesh(core_axis_name="c", subcore_axis_name="s")
    kern = pl.kernel(
        _seg_body,
        out_type=jax.ShapeDtypeStruct((NCORE, N, D), jnp.float32),
        mesh=mesh,
        scratch_types=[
            pltpu.VMEM((NCHUNK, CHUNK), jnp.int32),
            pltpu.VMEM((NCHUNK, CHUNK), jnp.int32),
            pltpu.VMEM((NCHUNK, CHUNK), jnp.float32),
            pltpu.VMEM((CHUNK, D), jnp.float32),
            pltpu.VMEM_SHARED((N, D), jnp.float32),
            pltpu.SemaphoreType.DMA,
        ],
    )
    return kern(x, src2, dst2, w2, zeros)


RB = 2000


def _lin_body(aL_ref, aN_ref, Wc_ref, Wh_ref, Wl_ref, bc_ref, bh_ref, bl_ref,
              o_ref):
    h = jnp.dot(aL_ref[...], Wc_ref[...], preferred_element_type=jnp.float32)
    h = h + 0.5 * jnp.dot(aN_ref[...], Wh_ref[...],
                          preferred_element_type=jnp.float32)
    h = h + (bc_ref[...] + 0.5 * bh_ref[...])
    o_ref[...] = (jnp.dot(h, Wl_ref[...], preferred_element_type=jnp.float32)
                  + bl_ref[...])


def _linear(aggL, aggN, Wc, Wh, Wl, bc, bh, bl):
    full = lambda i: (0, 0)
    return pl.pallas_call(
        _lin_body,
        grid=(N // RB,),
        in_specs=[
            pl.BlockSpec((RB, D), lambda i: (i, 0)),
            pl.BlockSpec((RB, D), lambda i: (i, 0)),
            pl.BlockSpec((D, D), full),
            pl.BlockSpec((D, D), full),
            pl.BlockSpec((D, D), full),
            pl.BlockSpec((1, D), full),
            pl.BlockSpec((1, D), full),
            pl.BlockSpec((1, D), full),
        ],
        out_specs=pl.BlockSpec((RB, D), lambda i: (i, 0)),
        out_shape=jax.ShapeDtypeStruct((N, D), jnp.float32),
    )(aggL, aggN, Wc, Wh, Wl, bc, bh, bl)


def kernel(x, edge_index_low, edge_weight_low, edge_index_high,
           edge_weight_high, edge_index_nd_low, edge_weight_nd_low,
           edge_index_nd_high, edge_weight_nd_high,
           W_conv, b_conv, W_hiconv, b_hiconv, W_lin, b_lin):
    src2 = jnp.stack([edge_index_low[0], edge_index_nd_low[0]])
    dst2 = jnp.stack([edge_index_low[1], edge_index_nd_low[1]])
    w2 = jnp.stack([edge_weight_low, edge_weight_nd_low])
    src2 = src2.reshape(NCORE, E // CHUNK, CHUNK)
    dst2 = dst2.reshape(NCORE, E // CHUNK, CHUNK)
    w2 = w2.reshape(NCORE, E // CHUNK, CHUNK)
    zeros = jnp.zeros((N, D), jnp.float32)

    agg = _sc_aggregate(x, src2, dst2, w2, zeros)
    return _linear(agg[0], agg[1], W_conv, W_hiconv, W_lin,
                   b_conv.reshape(1, D), b_hiconv.reshape(1, D),
                   b_lin.reshape(1, D))


# speedup vs baseline: 4.5550x; 4.5550x over previous
import dataclasses
import functools

import jax
import jax.numpy as jnp
from jax import lax
from jax.experimental import pallas as pl
from jax.experimental.pallas import tpu as pltpu
from jax.experimental.pallas import tpu_sc as plsc

N = 10000
E = 320000
D = 128
NCORE = 2
NSUB = 16
LANES = 16
CHUNK = 80
EPT = E // NSUB
NCHUNK = EPT // CHUNK
BATCH = 50
NBATCH = NCHUNK // BATCH
NP = 10112
ROWS_PT = NP // NSUB


def _seg_body(x_hbm, src_hbm, dst_hbm, w_hbm, zero_hbm, out_hbm,
              srcv, dstv, wv, rows, acc, sem):
    cid = lax.axis_index("c")
    sid = lax.axis_index("s")
    row0 = sid * ROWS_PT
    pltpu.sync_copy(zero_hbm.at[pl.ds(row0, ROWS_PT)],
                    acc.at[pl.ds(row0, ROWS_PT)])
    plsc.subcore_barrier()

    @pl.loop(0, NBATCH)
    def _(b):
        pltpu.sync_copy(src_hbm.at[cid, sid, b], srcv)
        pltpu.sync_copy(dst_hbm.at[cid, sid, b], dstv)
        pltpu.sync_copy(w_hbm.at[cid, sid, b], wv)

        @pl.loop(0, BATCH)
        def _(c):
            pltpu.async_copy(x_hbm.at[srcv.at[c]], rows, sem).wait()

            widx_c = jnp.full((LANES,), 0, jnp.int32) + c

            @pl.loop(0, CHUNK)
            def _(e):
                widx_e = jnp.full((LANES,), 0, jnp.int32) + e
                w = plsc.load_gather(wv, [widx_c, widx_e])
                for j in range(D // LANES):
                    sl = (e, pl.ds(j * LANES, LANES))
                    rows[sl] = rows[sl] * w

            pltpu.sync_copy(rows, acc.at[dstv.at[c]], add=True)

    plsc.subcore_barrier()
    pltpu.sync_copy(acc.at[pl.ds(row0, ROWS_PT)],
                    out_hbm.at[cid, pl.ds(row0, ROWS_PT)])


def _sc_aggregate(x, src2, dst2, w2, zeros):
    mesh = plsc.VectorSubcoreMesh(core_axis_name="c", subcore_axis_name="s")
    cp = pltpu.CompilerParams()
    if "needs_layout_passes" in pltpu.CompilerParams.__dataclass_fields__:
        cp = dataclasses.replace(cp, needs_layout_passes=False)
    kern = pl.kernel(
        _seg_body,
        out_type=jax.ShapeDtypeStruct((NCORE, NP, D), jnp.float32),
        mesh=mesh,
        scratch_types=[
            pltpu.VMEM((BATCH, CHUNK), jnp.int32),
            pltpu.VMEM((BATCH, CHUNK), jnp.int32),
            pltpu.VMEM((BATCH, CHUNK), jnp.float32),
            pltpu.VMEM((CHUNK, D), jnp.float32),
            pltpu.VMEM_SHARED((NP, D), jnp.float32),
            pltpu.SemaphoreType.DMA,
        ],
        compiler_params=cp,
    )
    return kern(x, src2, dst2, w2, zeros)


RB = 2000


def _lin_body(aL_ref, aN_ref, Wc_ref, Wh_ref, Wl_ref, bc_ref, bh_ref, bl_ref,
              o_ref):
    h = jnp.dot(aL_ref[...], Wc_ref[...], preferred_element_type=jnp.float32)
    h = h + 0.5 * jnp.dot(aN_ref[...], Wh_ref[...],
                          preferred_element_type=jnp.float32)
    h = h + (bc_ref[...] + 0.5 * bh_ref[...])
    o_ref[...] = (jnp.dot(h, Wl_ref[...], preferred_element_type=jnp.float32)
                  + bl_ref[...])


def _linear(aggL, aggN, Wc, Wh, Wl, bc, bh, bl):
    full = lambda i: (0, 0)
    return pl.pallas_call(
        _lin_body,
        grid=(N // RB,),
        in_specs=[
            pl.BlockSpec((RB, D), lambda i: (i, 0)),
            pl.BlockSpec((RB, D), lambda i: (i, 0)),
            pl.BlockSpec((D, D), full),
            pl.BlockSpec((D, D), full),
            pl.BlockSpec((D, D), full),
            pl.BlockSpec((1, D), full),
            pl.BlockSpec((1, D), full),
            pl.BlockSpec((1, D), full),
        ],
        out_specs=pl.BlockSpec((RB, D), lambda i: (i, 0)),
        out_shape=jax.ShapeDtypeStruct((N, D), jnp.float32),
    )(aggL, aggN, Wc, Wh, Wl, bc, bh, bl)


def kernel(x, edge_index_low, edge_weight_low, edge_index_high,
           edge_weight_high, edge_index_nd_low, edge_weight_nd_low,
           edge_index_nd_high, edge_weight_nd_high,
           W_conv, b_conv, W_hiconv, b_hiconv, W_lin, b_lin):
    src2 = jnp.stack([edge_index_low[0], edge_index_nd_low[0]])
    dst2 = jnp.stack([edge_index_low[1], edge_index_nd_low[1]])
    w2 = jnp.stack([edge_weight_low, edge_weight_nd_low])
    src2 = src2.reshape(NCORE, NSUB, NBATCH, BATCH, CHUNK)
    dst2 = dst2.reshape(NCORE, NSUB, NBATCH, BATCH, CHUNK)
    w2 = w2.reshape(NCORE, NSUB, NBATCH, BATCH, CHUNK)
    zeros = jnp.zeros((NP, D), jnp.float32)

    agg = _sc_aggregate(x, src2, dst2, w2, zeros)
    return _linear(agg[0, :N], agg[1, :N], W_conv, W_hiconv, W_lin,
                   b_conv.reshape(1, D), b_hiconv.reshape(1, D),
                   b_lin.reshape(1, D))


# speedup vs baseline: 6.9860x; 1.5337x over previous
import dataclasses
import functools

import jax
import jax.numpy as jnp
from jax import lax
from jax.experimental import pallas as pl
from jax.experimental.pallas import tpu as pltpu
from jax.experimental.pallas import tpu_sc as plsc

N = 10000
E = 320000
D = 128
NCORE = 2
NSUB = 16
LANES = 16
CHUNK = 80
EPT = E // NSUB
NCHUNK = EPT // CHUNK
BATCH = 50
NBATCH = NCHUNK // BATCH
NP = 10112
ROWS_PT = NP // NSUB


def _scale_rows(rows, wv, c):
    widx_c = jnp.full((LANES,), 0, jnp.int32) + c

    @pl.loop(0, CHUNK)
    def _(e):
        widx_e = jnp.full((LANES,), 0, jnp.int32) + e
        w = plsc.load_gather(wv, [widx_c, widx_e])
        for j in range(D // LANES):
            sl = (e, pl.ds(j * LANES, LANES))
            rows[sl] = rows[sl] * w


def _seg_body(x_hbm, src_hbm, dst_hbm, w_hbm, zero_hbm, out_hbm,
              srcv, dstv, wv, rows0, rows1, acc, sg0, sg1, ss0, ss1):
    cid = lax.axis_index("c")
    sid = lax.axis_index("s")
    row0 = sid * ROWS_PT
    pltpu.sync_copy(zero_hbm.at[pl.ds(row0, ROWS_PT)],
                    acc.at[pl.ds(row0, ROWS_PT)])
    plsc.subcore_barrier()

    @pl.loop(0, NBATCH)
    def _(b):
        pltpu.sync_copy(src_hbm.at[cid, sid, b], srcv)
        pltpu.sync_copy(dst_hbm.at[cid, sid, b], dstv)
        pltpu.sync_copy(w_hbm.at[cid, sid, b], wv)

        pltpu.async_copy(x_hbm.at[srcv.at[0]], rows0, sg0)
        pltpu.async_copy(x_hbm.at[srcv.at[1]], rows1, sg1)

        @pl.loop(0, BATCH, step=2)
        def _(c):
            pltpu.make_async_copy(x_hbm.at[srcv.at[c]], rows0, sg0).wait()
            _scale_rows(rows0, wv, c)
            s0 = pltpu.async_copy(rows0, acc.at[dstv.at[c]], ss0, add=True)

            pltpu.make_async_copy(x_hbm.at[srcv.at[c + 1]], rows1, sg1).wait()
            _scale_rows(rows1, wv, c + 1)
            s1 = pltpu.async_copy(rows1, acc.at[dstv.at[c + 1]], ss1,
                                  add=True)

            s0.wait()

            @pl.when(c + 2 < BATCH)
            def _():
                pltpu.async_copy(x_hbm.at[srcv.at[c + 2]], rows0, sg0)

            s1.wait()

            @pl.when(c + 3 < BATCH)
            def _():
                pltpu.async_copy(x_hbm.at[srcv.at[c + 3]], rows1, sg1)

    plsc.subcore_barrier()
    pltpu.sync_copy(acc.at[pl.ds(row0, ROWS_PT)],
                    out_hbm.at[cid, pl.ds(row0, ROWS_PT)])


def _sc_aggregate(x, src2, dst2, w2, zeros):
    mesh = plsc.VectorSubcoreMesh(core_axis_name="c", subcore_axis_name="s")
    cp = pltpu.CompilerParams()
    if "needs_layout_passes" in pltpu.CompilerParams.__dataclass_fields__:
        cp = dataclasses.replace(cp, needs_layout_passes=False)
    kern = pl.kernel(
        _seg_body,
        out_type=jax.ShapeDtypeStruct((NCORE, NP, D), jnp.float32),
        mesh=mesh,
        scratch_types=[
            pltpu.VMEM((BATCH, CHUNK), jnp.int32),
            pltpu.VMEM((BATCH, CHUNK), jnp.int32),
            pltpu.VMEM((BATCH, CHUNK), jnp.float32),
            pltpu.VMEM((CHUNK, D), jnp.float32),
            pltpu.VMEM((CHUNK, D), jnp.float32),
            pltpu.VMEM_SHARED((NP, D), jnp.float32),
            pltpu.SemaphoreType.DMA,
            pltpu.SemaphoreType.DMA,
            pltpu.SemaphoreType.DMA,
            pltpu.SemaphoreType.DMA,
        ],
        compiler_params=cp,
    )
    return kern(x, src2, dst2, w2, zeros)


RB = 2000


def _lin_body(aL_ref, aN_ref, Wc_ref, Wh_ref, Wl_ref, bc_ref, bh_ref, bl_ref,
              o_ref):
    h = jnp.dot(aL_ref[...], Wc_ref[...], preferred_element_type=jnp.float32)
    h = h + 0.5 * jnp.dot(aN_ref[...], Wh_ref[...],
                          preferred_element_type=jnp.float32)
    h = h + (bc_ref[...] + 0.5 * bh_ref[...])
    o_ref[...] = (jnp.dot(h, Wl_ref[...], preferred_element_type=jnp.float32)
                  + bl_ref[...])


def _linear(aggL, aggN, Wc, Wh, Wl, bc, bh, bl):
    full = lambda i: (0, 0)
    return pl.pallas_call(
        _lin_body,
        grid=(N // RB,),
        in_specs=[
            pl.BlockSpec((RB, D), lambda i: (i, 0)),
            pl.BlockSpec((RB, D), lambda i: (i, 0)),
            pl.BlockSpec((D, D), full),
            pl.BlockSpec((D, D), full),
            pl.BlockSpec((D, D), full),
            pl.BlockSpec((1, D), full),
            pl.BlockSpec((1, D), full),
            pl.BlockSpec((1, D), full),
        ],
        out_specs=pl.BlockSpec((RB, D), lambda i: (i, 0)),
        out_shape=jax.ShapeDtypeStruct((N, D), jnp.float32),
    )(aggL, aggN, Wc, Wh, Wl, bc, bh, bl)


def kernel(x, edge_index_low, edge_weight_low, edge_index_high,
           edge_weight_high, edge_index_nd_low, edge_weight_nd_low,
           edge_index_nd_high, edge_weight_nd_high,
           W_conv, b_conv, W_hiconv, b_hiconv, W_lin, b_lin):
    src2 = jnp.stack([edge_index_low[0], edge_index_nd_low[0]])
    dst2 = jnp.stack([edge_index_low[1], edge_index_nd_low[1]])
    w2 = jnp.stack([edge_weight_low, edge_weight_nd_low])
    src2 = src2.reshape(NCORE, NSUB, NBATCH, BATCH, CHUNK)
    dst2 = dst2.reshape(NCORE, NSUB, NBATCH, BATCH, CHUNK)
    w2 = w2.reshape(NCORE, NSUB, NBATCH, BATCH, CHUNK)
    zeros = jnp.zeros((NP, D), jnp.float32)

    agg = _sc_aggregate(x, src2, dst2, w2, zeros)
    return _linear(agg[0, :N], agg[1, :N], W_conv, W_hiconv, W_lin,
                   b_conv.reshape(1, D), b_hiconv.reshape(1, D),
                   b_lin.reshape(1, D))
